# baseline (speedup 1.0000x reference)
import jax
import jax.numpy as jnp
from jax.experimental import pallas as pl


def _rel_exp_body(x_ref, e_ref):
    x = x_ref[...]
    x_max = jnp.max(x, axis=-1, keepdims=True)
    e_ref[...] = jnp.exp(x - x_max)


def _rel_finish_body(e_ref, s_ref, prob_ref, score_ref, cls_ref):
    e = e_ref[...]
    s = s_ref[...]
    p = e / s
    q = p[:, 1:]
    m = jnp.max(q, axis=1)
    n_cls = q.shape[1]
    iota = jax.lax.broadcasted_iota(jnp.int32, q.shape, 1)
    idx = jnp.min(jnp.where(q == m[:, None], iota, n_cls), axis=1)
    prob_ref[...] = p
    score_ref[...] = m[:, None]
    cls_ref[...] = (idx + 1)[:, None]


def kernel(rel_logit, obj_logit, rel_pair_idx):
    N, C = rel_logit.shape

    obj_prob = jax.nn.softmax(obj_logit, axis=-1)
    obj_prob = obj_prob.at[:, 0].set(0.0)
    obj_scores = jnp.max(obj_prob[:, 1:], axis=1)
    obj_class = jnp.argmax(obj_prob[:, 1:], axis=1) + 1

    BLK = 2000
    e = pl.pallas_call(
        _rel_exp_body,
        grid=(N // BLK,),
        in_specs=[pl.BlockSpec((BLK, C), lambda i: (i, 0))],
        out_specs=pl.BlockSpec((BLK, C), lambda i: (i, 0)),
        out_shape=jax.ShapeDtypeStruct((N, C), jnp.float32),
    )(rel_logit)
    s = jnp.sum(e, axis=-1, keepdims=True)
    rel_prob, rel_scores, rel_class = pl.pallas_call(
        _rel_finish_body,
        grid=(N // BLK,),
        in_specs=[pl.BlockSpec((BLK, C), lambda i: (i, 0)),
                  pl.BlockSpec((BLK, 1), lambda i: (i, 0))],
        out_specs=(
            pl.BlockSpec((BLK, C), lambda i: (i, 0)),
            pl.BlockSpec((BLK, 1), lambda i: (i, 0)),
            pl.BlockSpec((BLK, 1), lambda i: (i, 0)),
        ),
        out_shape=(
            jax.ShapeDtypeStruct((N, C), jnp.float32),
            jax.ShapeDtypeStruct((N, 1), jnp.float32),
            jax.ShapeDtypeStruct((N, 1), jnp.int32),
        ),
    )(e, s)
    rel_scores = rel_scores.reshape(N)
    rel_class = rel_class.reshape(N)

    s0 = obj_scores[rel_pair_idx[:, 0]]
    s1 = obj_scores[rel_pair_idx[:, 1]]
    triple = rel_scores * s0 * s1
    sorting_idx = jnp.argsort(-triple)
    return (
        obj_class,
        obj_scores,
        rel_pair_idx[sorting_idx],
        rel_prob[sorting_idx],
        rel_class[sorting_idx],
    )


# speedup vs baseline: 1.1819x; 1.1819x over previous
import jax
import jax.numpy as jnp
from jax import lax
from jax.experimental import pallas as pl
from jax.experimental.pallas import tpu as pltpu
from jax.experimental.pallas import tpu_sc as plsc

N = 20000
C = 51
TW = 128
NPAD = 20480
CHUNK = 1280
OCHUNK = 640
TAIL = N - 31 * OCHUNK
RADIX = 2048
NHIST = 16 * RADIX


def _rel_exp_body(x_ref, e_ref):
    x = x_ref[...]
    x_max = jnp.max(x, axis=-1, keepdims=True)
    e_ref[...] = jnp.exp(x - x_max)


def _rel_finish_body(e_ref, s_ref, p0_ref, p1_ref, table_ref, score_ref):
    e = e_ref[...]
    s = s_ref[...]
    p = e / s
    q = p[:, 1:]
    m = jnp.max(q, axis=1)
    n_cls = q.shape[1]
    iota = lax.broadcasted_iota(jnp.int32, q.shape, 1)
    idx = jnp.min(jnp.where(q == m[:, None], iota, n_cls), axis=1)
    cls = (idx + 1)[:, None]
    bc = lambda a: lax.bitcast_convert_type(a, jnp.float32)
    zeros = jnp.zeros((p.shape[0], TW - C - 3), jnp.float32)
    table_ref[...] = jnp.concatenate(
        [p, bc(p0_ref[...]), bc(p1_ref[...]), bc(cls), zeros], axis=1)
    score_ref[...] = m[:, None]


def _sc_body(rs_ref, os_ref, p0_ref, p1_ref, table_ref,
             out_ref,
             spk_a, spi_a, spk_b, spi_b, sp_hist,
             v_scores, v_rs, v_i0, v_i1, v_key, v_idx, v_hist,
             v_t, v_c, v_next, v_pos, v_sidx, v_rows):
    s_id = lax.axis_index("s")
    c_id = lax.axis_index("c")
    base = s_id * CHUNK
    lane = lax.iota(jnp.int32, 16)
    zeros16 = jnp.zeros((16,), jnp.int32)

    pltpu.sync_copy(os_ref, v_scores)
    pltpu.sync_copy(rs_ref.at[pl.ds(base, CHUNK)], v_rs)
    pltpu.sync_copy(p0_ref.at[pl.ds(base, CHUNK)], v_i0)
    pltpu.sync_copy(p1_ref.at[pl.ds(base, CHUNK)], v_i1)

    def build_body(v, carry):
        sl = pl.ds(v * 16, 16)
        sa = plsc.load_gather(v_scores, [v_i0[sl]])
        sb = plsc.load_gather(v_scores, [v_i1[sl]])
        t = (v_rs[sl] * sa) * sb
        inv = ~plsc.bitcast(t, jnp.uint32)
        gidx = base + v * 16 + lane
        inv = jnp.where(gidx < N, inv, jnp.uint32(0xFFFFFFFF))
        v_key[sl] = inv
        v_idx[sl] = gidx
        return carry

    lax.fori_loop(0, CHUNK // 16, build_body, 0)
    pltpu.sync_copy(v_key, spk_a.at[pl.ds(base, CHUNK)])
    pltpu.sync_copy(v_idx, spi_a.at[pl.ds(base, CHUNK)])
    plsc.subcore_barrier()

    for p, (src_k, src_i, dst_k, dst_i) in enumerate(
            [(spk_a, spi_a, spk_b, spi_b),
             (spk_b, spi_b, spk_a, spi_a),
             (spk_a, spi_a, spk_b, spi_b)]):
        shift = jnp.uint32(p * 11)
        mask = jnp.uint32(RADIX - 1)

        pltpu.sync_copy(src_k.at[pl.ds(base, CHUNK)], v_key)

        def zero_body(i, carry):
            v_hist[pl.ds(i * 16, 16)] = zeros16
            return carry

        lax.fori_loop(0, RADIX // 16, zero_body, 0)

        def hist_body(v, carry):
            k = v_key[pl.ds(v * 16, 16)]
            d = ((k >> shift) & mask).astype(jnp.int32)
            occ, last = plsc.scan_count(d)
            plsc.addupdate_scatter(v_hist, [d], occ, mask=last)
            return carry

        lax.fori_loop(0, CHUNK // 16, hist_body, 0)
        pltpu.sync_copy(v_hist, sp_hist.at[pl.ds(s_id * RADIX, RADIX)])
        plsc.subcore_barrier()

        def zero_tc_body(i, carry):
            sl = pl.ds(i * 16, 16)
            v_t[sl] = zeros16
            v_c[sl] = zeros16
            return carry

        lax.fori_loop(0, RADIX // 16, zero_tc_body, 0)

        for l in range(16):
            pltpu.sync_copy(sp_hist.at[pl.ds(l * RADIX, RADIX)], v_hist)

            def scan_body(dc, carry, l=l):
                sl = pl.ds(dc * 16, 16)
                g = v_hist[sl]
                v_t[sl] = v_t[sl] + g
                v_c[sl] = v_c[sl] + jnp.where(l < s_id, g, 0)
                return carry

            lax.fori_loop(0, RADIX // 16, scan_body, 0)

        def prefix_body(dc, carry):
            sl = pl.ds(dc * 16, 16)
            t16 = v_t[sl]
            incl = plsc.cumsum(t16)
            v_next[sl] = v_c[sl] + (incl - t16) + carry
            return carry + jnp.sum(t16)

        lax.fori_loop(0, RADIX // 16, prefix_body, jnp.int32(0))

        pltpu.sync_copy(src_i.at[pl.ds(base, CHUNK)], v_idx)

        def rank_body(v, carry):
            sl = pl.ds(v * 16, 16)
            k = v_key[sl]
            d = ((k >> shift) & mask).astype(jnp.int32)
            nx = plsc.load_gather(v_next, [d])
            occ, last = plsc.scan_count(d)
            v_pos[v >> 3, pl.ds((v & 7) * 16, 16)] = nx + occ - 1
            plsc.store_scatter(v_next, [d], nx + occ, mask=last)
            return carry

        lax.fori_loop(0, CHUNK // 16, rank_body, 0)

        for j in range(CHUNK // 128):
            sl = pl.ds(j * 128, 128)
            pltpu.sync_copy(v_key.at[sl], dst_k.at[v_pos.at[j]])
            pltpu.sync_copy(v_idx.at[sl], dst_i.at[v_pos.at[j]])
        plsc.subcore_barrier()

    wid = c_id * 16 + s_id
    obase = wid * OCHUNK
    pltpu.sync_copy(spi_b.at[pl.ds(obase, OCHUNK)], v_sidx)

    def clamp_body(v, carry):
        sl = pl.ds(v * 16, 16)
        v_sidx[sl] = jnp.minimum(v_sidx[sl], N - 1)
        return carry

    lax.fori_loop(0, OCHUNK // 16, clamp_body, 0)

    for j in range(OCHUNK // 128):
        pltpu.sync_copy(table_ref.at[v_sidx.at[pl.ds(j * 128, 128)]],
                        v_rows.at[pl.ds(j * 128, 128)])

    @pl.when(wid < 31)
    def _():
        pltpu.sync_copy(v_rows.at[pl.ds(0, OCHUNK)],
                        out_ref.at[pl.ds(obase, OCHUNK)])

    @pl.when(wid == 31)
    def _():
        pltpu.sync_copy(v_rows.at[pl.ds(0, TAIL)],
                        out_ref.at[pl.ds(31 * OCHUNK, TAIL)])


def kernel(rel_logit, obj_logit, rel_pair_idx):
    n, c = rel_logit.shape

    obj_prob = jax.nn.softmax(obj_logit, axis=-1)
    obj_prob = obj_prob.at[:, 0].set(0.0)
    obj_scores = jnp.max(obj_prob[:, 1:], axis=1)
    obj_class = jnp.argmax(obj_prob[:, 1:], axis=1) + 1

    BLK = 2000
    e = pl.pallas_call(
        _rel_exp_body,
        grid=(n // BLK,),
        in_specs=[pl.BlockSpec((BLK, c), lambda i: (i, 0))],
        out_specs=pl.BlockSpec((BLK, c), lambda i: (i, 0)),
        out_shape=jax.ShapeDtypeStruct((n, c), jnp.float32),
    )(rel_logit)
    s = jnp.sum(e, axis=-1, keepdims=True)
    table, rel_scores = pl.pallas_call(
        _rel_finish_body,
        grid=(n // BLK,),
        in_specs=[pl.BlockSpec((BLK, c), lambda i: (i, 0)),
                  pl.BlockSpec((BLK, 1), lambda i: (i, 0)),
                  pl.BlockSpec((BLK, 1), lambda i: (i, 0)),
                  pl.BlockSpec((BLK, 1), lambda i: (i, 0))],
        out_specs=(
            pl.BlockSpec((BLK, TW), lambda i: (i, 0)),
            pl.BlockSpec((BLK, 1), lambda i: (i, 0)),
        ),
        out_shape=(
            jax.ShapeDtypeStruct((n, TW), jnp.float32),
            jax.ShapeDtypeStruct((n, 1), jnp.float32),
        ),
    )(e, s, rel_pair_idx[:, 0:1], rel_pair_idx[:, 1:2])
    rel_scores = rel_scores.reshape(n)

    rs_pad = jnp.pad(rel_scores, (0, NPAD - n))
    rpi_pad = jnp.pad(rel_pair_idx, ((0, NPAD - n), (0, 0)))
    col0 = rpi_pad[:, 0]
    col1 = rpi_pad[:, 1]
    os_pad = jnp.pad(obj_scores, (0, 1024 - obj_scores.shape[0]))

    mesh = plsc.VectorSubcoreMesh(core_axis_name="c", subcore_axis_name="s")
    out_table = pl.kernel(
        _sc_body,
        out_type=jax.ShapeDtypeStruct((N, TW), jnp.float32),
        mesh=mesh,
        compiler_params=pltpu.CompilerParams(needs_layout_passes=False),
        scratch_types=[
            pltpu.VMEM_SHARED((NPAD,), jnp.uint32),
            pltpu.VMEM_SHARED((NPAD,), jnp.int32),
            pltpu.VMEM_SHARED((NPAD,), jnp.uint32),
            pltpu.VMEM_SHARED((NPAD,), jnp.int32),
            pltpu.VMEM_SHARED((NHIST,), jnp.int32),
            pltpu.VMEM((1024,), jnp.float32),
            pltpu.VMEM((CHUNK,), jnp.float32),
            pltpu.VMEM((CHUNK,), jnp.int32),
            pltpu.VMEM((CHUNK,), jnp.int32),
            pltpu.VMEM((CHUNK,), jnp.uint32),
            pltpu.VMEM((CHUNK,), jnp.int32),
            pltpu.VMEM((RADIX,), jnp.int32),
            pltpu.VMEM((RADIX,), jnp.int32),
            pltpu.VMEM((RADIX,), jnp.int32),
            pltpu.VMEM((RADIX,), jnp.int32),
            pltpu.VMEM((CHUNK // 128, 128), jnp.int32),
            pltpu.VMEM((OCHUNK,), jnp.int32),
            pltpu.VMEM((OCHUNK, TW), jnp.float32),
        ],
    )(rs_pad, os_pad, col0, col1, table)

    return (
        obj_class,
        obj_scores,
        lax.bitcast_convert_type(out_table[:, C:C + 2], jnp.int32),
        out_table[:, :C],
        lax.bitcast_convert_type(out_table[:, C + 2], jnp.int32),
    )


# speedup vs baseline: 1.3494x; 1.1417x over previous
import jax
import jax.numpy as jnp
from jax import lax
from jax.experimental import pallas as pl
from jax.experimental.pallas import tpu as pltpu
from jax.experimental.pallas import tpu_sc as plsc

N = 20000
C = 51
TW = 128
NPAD = 20480
CHUNK = 1280
OCHUNK = 640
TAIL = N - 31 * OCHUNK
RADIX = 2048
NHIST = 16 * RADIX


def _rel_exp_body(x_ref, e_ref):
    x = x_ref[...]
    x_max = jnp.max(x, axis=-1, keepdims=True)
    e_ref[...] = jnp.exp(x - x_max)


def _rel_finish_body(e_ref, s_ref, p0_ref, p1_ref, table_ref, score_ref):
    e = e_ref[...]
    s = s_ref[...]
    p = e / s
    q = p[:, 1:]
    m = jnp.max(q, axis=1)
    n_cls = q.shape[1]
    iota = lax.broadcasted_iota(jnp.int32, q.shape, 1)
    idx = jnp.min(jnp.where(q == m[:, None], iota, n_cls), axis=1)
    cls = (idx + 1)[:, None]
    bc = lambda a: lax.bitcast_convert_type(a, jnp.float32)
    zeros = jnp.zeros((p.shape[0], TW - C - 3), jnp.float32)
    table_ref[...] = jnp.concatenate(
        [p, bc(p0_ref[...]), bc(p1_ref[...]), bc(cls), zeros], axis=1)
    score_ref[...] = m[:, None]


def _sc_body(rs_ref, os_ref, p0_ref, p1_ref, table_ref,
             out_ref,
             spk_a, spi_a, spk_b, spi_b, sp_hist,
             v_scores, v_rs, v_i0, v_i1, v_key, v_idx, v_hist, v_ghist,
             v_t, v_c, v_next, v_pos, v_sidx, v_rows, sem_in, sem_s, sem_g):
    s_id = lax.axis_index("s")
    c_id = lax.axis_index("c")
    base = s_id * CHUNK
    lane = lax.iota(jnp.int32, 16)
    zeros16 = jnp.zeros((16,), jnp.int32)

    loads = [
        pltpu.async_copy(os_ref, v_scores, sem_in),
        pltpu.async_copy(rs_ref.at[pl.ds(base, CHUNK)], v_rs, sem_in),
        pltpu.async_copy(p0_ref.at[pl.ds(base, CHUNK)], v_i0, sem_in),
        pltpu.async_copy(p1_ref.at[pl.ds(base, CHUNK)], v_i1, sem_in),
    ]
    for ld in loads:
        ld.wait()

    def build_body(v, carry):
        sl = pl.ds(v * 16, 16)
        sa = plsc.load_gather(v_scores, [v_i0[sl]])
        sb = plsc.load_gather(v_scores, [v_i1[sl]])
        t = (v_rs[sl] * sa) * sb
        inv = ~plsc.bitcast(t, jnp.uint32)
        gidx = base + v * 16 + lane
        inv = jnp.where(gidx < N, inv, jnp.uint32(0xFFFFFFFF))
        v_key[sl] = inv
        v_idx[sl] = gidx
        return carry

    lax.fori_loop(0, CHUNK // 16, build_body, 0)
    pltpu.sync_copy(v_key, spk_a.at[pl.ds(base, CHUNK)])
    pltpu.sync_copy(v_idx, spi_a.at[pl.ds(base, CHUNK)])
    plsc.subcore_barrier()

    for p, (src_k, src_i, dst_k, dst_i) in enumerate(
            [(spk_a, spi_a, spk_b, spi_b),
             (spk_b, spi_b, spk_a, spi_a),
             (spk_a, spi_a, spk_b, spi_b)]):
        shift = jnp.uint32(p * 11)
        mask = jnp.uint32(RADIX - 1)

        pltpu.sync_copy(src_k.at[pl.ds(base, CHUNK)], v_key)

        def zero_body(i, carry):
            v_hist[pl.ds(i * 16, 16)] = zeros16
            return carry

        lax.fori_loop(0, RADIX // 16, zero_body, 0)

        def hist_body(v, carry):
            k = v_key[pl.ds(v * 16, 16)]
            d = ((k >> shift) & mask).astype(jnp.int32)
            occ, last = plsc.scan_count(d)
            plsc.addupdate_scatter(v_hist, [d], occ, mask=last)
            return carry

        lax.fori_loop(0, CHUNK // 16, hist_body, 0)
        pltpu.sync_copy(v_hist, sp_hist.at[pl.ds(s_id * RADIX, RADIX)])
        plsc.subcore_barrier()

        pltpu.sync_copy(sp_hist, v_ghist)

        def scan_body(dc, carry):
            sl = pl.ds(dc * 16, 16)
            acc_t = v_ghist[pl.ds(dc * 16, 16)]
            acc_c = zeros16
            for l in range(1, 16):
                g = v_ghist[pl.ds(l * RADIX + dc * 16, 16)]
                acc_t = acc_t + g
                acc_c = acc_c + jnp.where(l <= s_id - 1, g, 0)
            v_t[sl] = acc_t
            v_c[sl] = acc_c + jnp.where(s_id > 0, v_ghist[pl.ds(dc * 16, 16)], 0)
            return carry

        lax.fori_loop(0, RADIX // 16, scan_body, 0)

        def prefix_body(dc, carry):
            sl = pl.ds(dc * 16, 16)
            t16 = v_t[sl]
            incl = plsc.cumsum(t16)
            v_next[sl] = v_c[sl] + (incl - t16) + carry
            return carry + jnp.sum(t16)

        lax.fori_loop(0, RADIX // 16, prefix_body, jnp.int32(0))

        pltpu.sync_copy(src_i.at[pl.ds(base, CHUNK)], v_idx)

        def rank_body(v, carry):
            sl = pl.ds(v * 16, 16)
            k = v_key[sl]
            d = ((k >> shift) & mask).astype(jnp.int32)
            nx = plsc.load_gather(v_next, [d])
            occ, last = plsc.scan_count(d)
            v_pos[v >> 3, pl.ds((v & 7) * 16, 16)] = nx + occ - 1
            plsc.store_scatter(v_next, [d], nx + occ, mask=last)
            return carry

        lax.fori_loop(0, CHUNK // 16, rank_body, 0)

        scats = []
        for j in range(CHUNK // 128):
            sl = pl.ds(j * 128, 128)
            scats.append(pltpu.async_copy(v_key.at[sl], dst_k.at[v_pos.at[j]], sem_s))
            scats.append(pltpu.async_copy(v_idx.at[sl], dst_i.at[v_pos.at[j]], sem_s))
        for sc in scats:
            sc.wait()
        plsc.subcore_barrier()

    wid = c_id * 16 + s_id
    obase = wid * OCHUNK
    pltpu.sync_copy(spi_b.at[pl.ds(obase, OCHUNK)], v_sidx)

    def clamp_body(v, carry):
        sl = pl.ds(v * 16, 16)
        v_sidx[sl] = jnp.minimum(v_sidx[sl], N - 1)
        return carry

    lax.fori_loop(0, OCHUNK // 16, clamp_body, 0)

    NJ = OCHUNK // 128
    TAIL2 = N % 128

    def start_gather(j, buf):
        return pltpu.async_copy(
            table_ref.at[v_sidx.at[pl.ds(j * 128, 128)]], v_rows.at[buf], sem_g)

    g = start_gather(0, 0)
    for j in range(NJ):
        g.wait()
        if j + 1 < NJ:
            g = start_gather(j + 1, (j + 1) % 2)
        off = obase + j * 128

        @pl.when(off + 128 <= N)
        def _(j=j):
            pltpu.sync_copy(v_rows.at[j % 2], out_ref.at[pl.ds(off, 128)])

        @pl.when(jnp.logical_and(off < N, off + 128 > N))
        def _(j=j):
            pltpu.sync_copy(v_rows.at[j % 2, pl.ds(0, TAIL2)],
                            out_ref.at[pl.ds(off, TAIL2)])


def kernel(rel_logit, obj_logit, rel_pair_idx):
    n, c = rel_logit.shape

    obj_prob = jax.nn.softmax(obj_logit, axis=-1)
    obj_prob = obj_prob.at[:, 0].set(0.0)
    obj_scores = jnp.max(obj_prob[:, 1:], axis=1)
    obj_class = jnp.argmax(obj_prob[:, 1:], axis=1) + 1

    BLK = 2000
    e = pl.pallas_call(
        _rel_exp_body,
        grid=(n // BLK,),
        in_specs=[pl.BlockSpec((BLK, c), lambda i: (i, 0))],
        out_specs=pl.BlockSpec((BLK, c), lambda i: (i, 0)),
        out_shape=jax.ShapeDtypeStruct((n, c), jnp.float32),
    )(rel_logit)
    s = jnp.sum(e, axis=-1, keepdims=True)
    table, rel_scores = pl.pallas_call(
        _rel_finish_body,
        grid=(n // BLK,),
        in_specs=[pl.BlockSpec((BLK, c), lambda i: (i, 0)),
                  pl.BlockSpec((BLK, 1), lambda i: (i, 0)),
                  pl.BlockSpec((BLK, 1), lambda i: (i, 0)),
                  pl.BlockSpec((BLK, 1), lambda i: (i, 0))],
        out_specs=(
            pl.BlockSpec((BLK, TW), lambda i: (i, 0)),
            pl.BlockSpec((BLK, 1), lambda i: (i, 0)),
        ),
        out_shape=(
            jax.ShapeDtypeStruct((n, TW), jnp.float32),
            jax.ShapeDtypeStruct((n, 1), jnp.float32),
        ),
    )(e, s, rel_pair_idx[:, 0:1], rel_pair_idx[:, 1:2])
    rel_scores = rel_scores.reshape(n)

    rs_pad = jnp.pad(rel_scores, (0, NPAD - n))
    rpi_pad = jnp.pad(rel_pair_idx, ((0, NPAD - n), (0, 0)))
    col0 = rpi_pad[:, 0]
    col1 = rpi_pad[:, 1]
    os_pad = jnp.pad(obj_scores, (0, 1024 - obj_scores.shape[0]))

    mesh = plsc.VectorSubcoreMesh(core_axis_name="c", subcore_axis_name="s")
    out_table = pl.kernel(
        _sc_body,
        out_type=jax.ShapeDtypeStruct((N, TW), jnp.float32),
        mesh=mesh,
        compiler_params=pltpu.CompilerParams(needs_layout_passes=False),
        scratch_types=[
            pltpu.VMEM_SHARED((NPAD,), jnp.uint32),
            pltpu.VMEM_SHARED((NPAD,), jnp.int32),
            pltpu.VMEM_SHARED((NPAD,), jnp.uint32),
            pltpu.VMEM_SHARED((NPAD,), jnp.int32),
            pltpu.VMEM_SHARED((NHIST,), jnp.int32),
            pltpu.VMEM((1024,), jnp.float32),
            pltpu.VMEM((CHUNK,), jnp.float32),
            pltpu.VMEM((CHUNK,), jnp.int32),
            pltpu.VMEM((CHUNK,), jnp.int32),
            pltpu.VMEM((CHUNK,), jnp.uint32),
            pltpu.VMEM((CHUNK,), jnp.int32),
            pltpu.VMEM((RADIX,), jnp.int32),
            pltpu.VMEM((NHIST,), jnp.int32),
            pltpu.VMEM((RADIX,), jnp.int32),
            pltpu.VMEM((RADIX,), jnp.int32),
            pltpu.VMEM((RADIX,), jnp.int32),
            pltpu.VMEM((CHUNK // 128, 128), jnp.int32),
            pltpu.VMEM((OCHUNK,), jnp.int32),
            pltpu.VMEM((2, 128, TW), jnp.float32),
            pltpu.SemaphoreType.DMA,
            pltpu.SemaphoreType.DMA,
            pltpu.SemaphoreType.DMA,
        ],
    )(rs_pad, os_pad, col0, col1, table)

    return (
        obj_class,
        obj_scores,
        lax.bitcast_convert_type(out_table[:, C:C + 2], jnp.int32),
        out_table[:, :C],
        lax.bitcast_convert_type(out_table[:, C + 2], jnp.int32),
    )
